# baseline (speedup 1.0000x reference)
import functools

import jax
import jax.numpy as jnp
from jax import lax
from jax.experimental import pallas as pl
from jax.experimental.pallas import tpu as pltpu
from jax.experimental.pallas import tpu_sc as plsc

N = 100000
E = 6400000
NPAD = 100096
SLICE = NPAD // 16
PIECES = ((0, 1568), (1568, 1568), (3136, 1568), (4704, 1552))
PBUF = 1568
G = 120
NCHUNK = 13
CH = G * 128


@functools.lru_cache(maxsize=None)
def _mesh():
    return plsc.VectorSubcoreMesh(core_axis_name="c", subcore_axis_name="s",
                                  num_cores=2, num_subcores=16)


def _edge_pass(sub_ref, table, acc, src_b, dst_b, val_b,
               sem_i0, sem_i1, sem_g, sem_s, w):
    gbase = (w * 195 + jnp.minimum(w, 10)) * 8

    def idx_copies(i, p):
        e0 = (gbase + i * G) * 128
        sem = sem_i0 if p == 0 else sem_i1
        return (pltpu.make_async_copy(sub_ref.at[0, pl.ds(e0, CH)],
                                      src_b.at[p], sem),
                pltpu.make_async_copy(sub_ref.at[1, pl.ds(e0, CH)],
                                      dst_b.at[p], sem))

    def gather_copy(p):
        return pltpu.make_async_copy(table.at[src_b.at[p]], val_b.at[p],
                                     sem_g)

    par = (w % 2) * NPAD

    def acc_half():
        return acc.at[pl.ds(par, NPAD)]

    def scatter_copy(p):
        return pltpu.make_async_copy(val_b.at[p], acc_half().at[dst_b.at[p]],
                                     sem_s)

    for cpy in idx_copies(0, 0):
        cpy.start()
    for i in range(NCHUNK):
        p = i % 2
        for cpy in idx_copies(i, p):
            cpy.wait()
        gather_copy(p).start()
        if i >= 1:
            scatter_copy(1 - p).wait()
        if i + 1 < NCHUNK:
            for cpy in idx_copies(i + 1, 1 - p):
                cpy.start()
        gather_copy(p).wait()
        pltpu.async_copy(val_b.at[p], acc_half().at[dst_b.at[p]], sem_s,
                         add=True)
    scatter_copy((NCHUNK - 1) % 2).wait()

    @pl.when(w < 10)
    def _tail():
        e0 = (gbase + NCHUNK * G) * 128
        pltpu.sync_copy(sub_ref.at[0, pl.ds(e0, 1024)],
                        src_b.at[0, pl.ds(0, 1024)])
        pltpu.sync_copy(sub_ref.at[1, pl.ds(e0, 1024)],
                        dst_b.at[0, pl.ds(0, 1024)])
        pltpu.async_copy(table.at[src_b.at[0, pl.ds(0, 1024)]],
                         val_b.at[0, pl.ds(0, 1024)], sem_g).wait()
        pltpu.sync_copy(val_b.at[0, pl.ds(0, 1024)],
                        acc_half().at[dst_b.at[0, pl.ds(0, 1024)]], add=True)


def _zero_acc_slice(acc, pa, sl):
    def zbody(j, _):
        pa[pl.ds(j * 16, 16)] = jnp.zeros((16,), jnp.float32)
        return 0

    lax.fori_loop(0, PBUF // 16, zbody, 0)
    for off, ln in PIECES:
        pltpu.sync_copy(pa.at[pl.ds(0, ln)], acc.at[pl.ds(sl + off, ln)])
        pltpu.sync_copy(pa.at[pl.ds(0, ln)],
                        acc.at[pl.ds(NPAD + sl + off, ln)])


def _dump_acc_slice(acc, pa, pb, sl, out_ref, c):
    for off, ln in PIECES:
        pltpu.sync_copy(acc.at[pl.ds(sl + off, ln)], pa.at[pl.ds(0, ln)])
        pltpu.sync_copy(acc.at[pl.ds(NPAD + sl + off, ln)], pb.at[pl.ds(0, ln)])

        def abody(j, _):
            ix = pl.ds(j * 16, 16)
            pa[ix] = pa[ix] + pb[ix]
            return 0

        lax.fori_loop(0, ln // 16, abody, 0)
        pltpu.sync_copy(pa.at[pl.ds(0, ln)],
                        out_ref.at[pl.ds(c * NPAD + sl + off, ln)])


def _layer1_body(sub_ref, feat_ref, norm_ref, out_ref,
                 table_sp, acc, pa, pb, src_b, dst_b, val_b,
                 sem_i0, sem_i1, sem_g, sem_s):
    c = lax.axis_index("c")
    s = lax.axis_index("s")
    w = c * 16 + s
    sl = s * SLICE
    for off, ln in PIECES:
        pltpu.sync_copy(feat_ref.at[pl.ds(sl + off, ln)], pa.at[pl.ds(0, ln)])
        pltpu.sync_copy(norm_ref.at[pl.ds(sl + off, ln)], pb.at[pl.ds(0, ln)])

        def mbody(j, _):
            ix = pl.ds(j * 16, 16)
            pa[ix] = pa[ix] * pb[ix]
            return 0

        lax.fori_loop(0, ln // 16, mbody, 0)
        pltpu.sync_copy(pa.at[pl.ds(0, ln)], table_sp.at[pl.ds(sl + off, ln)])
    _zero_acc_slice(acc, pa, sl)
    plsc.subcore_barrier()

    _edge_pass(sub_ref, table_sp, acc, src_b, dst_b, val_b,
               sem_i0, sem_i1, sem_g, sem_s, w)

    plsc.subcore_barrier()
    _dump_acc_slice(acc, pa, pb, sl, out_ref, c)


def _layer2_body(sub_ref, p_ref, norm_ref, w_ref, out_ref,
                 table_sp, acc, pa, pb, pc, wb, src_b, dst_b, val_b,
                 sem_i0, sem_i1, sem_g, sem_s):
    c = lax.axis_index("c")
    s = lax.axis_index("s")
    w = c * 16 + s
    sl = s * SLICE
    pltpu.sync_copy(w_ref, wb)
    wv = [wb[pl.ds(k * 16, 16)] for k in range(9)]
    for off, ln in PIECES:
        pltpu.sync_copy(p_ref.at[pl.ds(sl + off, ln)], pa.at[pl.ds(0, ln)])
        pltpu.sync_copy(p_ref.at[pl.ds(NPAD + sl + off, ln)],
                        pb.at[pl.ds(0, ln)])
        pltpu.sync_copy(norm_ref.at[pl.ds(sl + off, ln)], pc.at[pl.ds(0, ln)])

        def sbody(j, _):
            ix = pl.ds(j * 16, 16)
            nv = pc[ix]
            t = (pa[ix] + pb[ix]) * nv
            sv = jnp.zeros((16,), jnp.float32)
            for k in range(3):
                sv = sv + jnp.maximum(t * wv[k] + wv[3 + k], 0.0) * wv[6 + k]
            pa[ix] = sv * nv
            return 0

        lax.fori_loop(0, ln // 16, sbody, 0)
        pltpu.sync_copy(pa.at[pl.ds(0, ln)], table_sp.at[pl.ds(sl + off, ln)])
    _zero_acc_slice(acc, pa, sl)
    plsc.subcore_barrier()

    _edge_pass(sub_ref, table_sp, acc, src_b, dst_b, val_b,
               sem_i0, sem_i1, sem_g, sem_s, w)

    plsc.subcore_barrier()
    _dump_acc_slice(acc, pa, pb, sl, out_ref, c)


def _fin_body(p_ref, n_ref, b_ref, o_ref):
    o_ref[...] = jnp.maximum((p_ref[0] + p_ref[1]) * n_ref[...] + b_ref[...], 0.0)


_finalize = pl.pallas_call(
    _fin_body,
    out_shape=jax.ShapeDtypeStruct((NPAD // 128, 128), jnp.float32),
)


@functools.lru_cache(maxsize=None)
def _build_layers():
    common = [
        pltpu.MemorySpace.VMEM_SHARED((NPAD,), jnp.float32),
        pltpu.MemorySpace.VMEM_SHARED((2 * NPAD,), jnp.float32),
        pltpu.MemorySpace.VMEM((PBUF,), jnp.float32),
        pltpu.MemorySpace.VMEM((PBUF,), jnp.float32),
    ]
    bufs = [
        pltpu.MemorySpace.VMEM((2, CH), jnp.int32),
        pltpu.MemorySpace.VMEM((2, CH), jnp.int32),
        pltpu.MemorySpace.VMEM((2, CH), jnp.float32),
        pltpu.SemaphoreType.DMA,
        pltpu.SemaphoreType.DMA,
        pltpu.SemaphoreType.DMA,
        pltpu.SemaphoreType.DMA,
    ]
    cp = pltpu.CompilerParams(use_tc_tiling_on_sc=False)
    l1 = pl.kernel(
        _layer1_body,
        compiler_params=cp,
        out_type=jax.ShapeDtypeStruct((2 * NPAD,), jnp.float32),
        mesh=_mesh(),
        scratch_types=common + bufs,
    )
    l2 = pl.kernel(
        _layer2_body,
        compiler_params=cp,
        out_type=jax.ShapeDtypeStruct((2 * NPAD,), jnp.float32),
        mesh=_mesh(),
        scratch_types=common + [
            pltpu.MemorySpace.VMEM((PBUF,), jnp.float32),
            pltpu.MemorySpace.VMEM((144,), jnp.float32),
        ] + bufs,
    )
    return l1, l2


def kernel(subgraph, feat, norm, W1, b1, W2, b2):
    _l1, _l2 = _build_layers()
    featp = jnp.pad(feat.reshape(N), (0, NPAD - N))
    normp = jnp.pad(norm.reshape(N), (0, NPAD - N))
    p1 = _l1(subgraph, featp, normp)
    wtab = jnp.concatenate([
        jnp.broadcast_to(W1.reshape(3, 1), (3, 16)).reshape(-1),
        jnp.broadcast_to(b1.reshape(3, 1), (3, 16)).reshape(-1),
        jnp.broadcast_to(W2.reshape(3, 1), (3, 16)).reshape(-1),
    ])
    p2 = _l2(subgraph, p1, normp, wtab)
    b2t = jnp.broadcast_to(b2.reshape(1, 1), (1, 1))
    out = _finalize(p2.reshape(2, NPAD // 128, 128),
                    normp.reshape(NPAD // 128, 128), b2t)
    return out.reshape(NPAD)[:N].reshape(N, 1)


# speedup vs baseline: 1.0185x; 1.0185x over previous
import functools

import jax
import jax.numpy as jnp
from jax import lax
from jax.experimental import pallas as pl
from jax.experimental.pallas import tpu as pltpu
from jax.experimental.pallas import tpu_sc as plsc

N = 100000
E = 6400000
NPAD = 100096
SLICE = NPAD // 16
PIECES = ((0, 1568), (1568, 1568), (3136, 1568), (4704, 1552))
PBUF = 1568
G = 120
NCHUNK = 13
CH = G * 128


@functools.lru_cache(maxsize=None)
def _mesh():
    return plsc.VectorSubcoreMesh(core_axis_name="c", subcore_axis_name="s",
                                  num_cores=2, num_subcores=16)


def _edge_pass(sub_ref, table, acc, src_b, dst_b, val_b,
               sem_i0, sem_i1, sem_g, sem_s, w):
    gbase = (w * 195 + jnp.minimum(w, 10)) * 8

    def idx_copies(i, p):
        e0 = (gbase + i * G) * 128
        sem = sem_i0 if p == 0 else sem_i1
        return (pltpu.make_async_copy(sub_ref.at[0, pl.ds(e0, CH)],
                                      src_b.at[p], sem),
                pltpu.make_async_copy(sub_ref.at[1, pl.ds(e0, CH)],
                                      dst_b.at[p], sem))

    def gather_copy(p):
        return pltpu.make_async_copy(table.at[src_b.at[p]], val_b.at[p],
                                     sem_g)

    def scatter_copy(p):
        return pltpu.make_async_copy(val_b.at[p], acc.at[dst_b.at[p]],
                                     sem_s)

    for cpy in idx_copies(0, 0):
        cpy.start()
    for i in range(NCHUNK):
        p = i % 2
        for cpy in idx_copies(i, p):
            cpy.wait()
        gather_copy(p).start()
        if i >= 1:
            scatter_copy(1 - p).wait()
        if i + 1 < NCHUNK:
            for cpy in idx_copies(i + 1, 1 - p):
                cpy.start()
        gather_copy(p).wait()
        pltpu.async_copy(val_b.at[p], acc.at[dst_b.at[p]], sem_s, add=True)
    scatter_copy((NCHUNK - 1) % 2).wait()

    @pl.when(w < 10)
    def _tail():
        e0 = (gbase + NCHUNK * G) * 128
        pltpu.sync_copy(sub_ref.at[0, pl.ds(e0, 1024)],
                        src_b.at[0, pl.ds(0, 1024)])
        pltpu.sync_copy(sub_ref.at[1, pl.ds(e0, 1024)],
                        dst_b.at[0, pl.ds(0, 1024)])
        pltpu.async_copy(table.at[src_b.at[0, pl.ds(0, 1024)]],
                         val_b.at[0, pl.ds(0, 1024)], sem_g).wait()
        pltpu.sync_copy(val_b.at[0, pl.ds(0, 1024)],
                        acc.at[dst_b.at[0, pl.ds(0, 1024)]], add=True)


def _zero_acc_slice(acc, pa, sl):
    def zbody(j, _):
        pa[pl.ds(j * 16, 16)] = jnp.zeros((16,), jnp.float32)
        return 0

    lax.fori_loop(0, PBUF // 16, zbody, 0)
    for off, ln in PIECES:
        pltpu.sync_copy(pa.at[pl.ds(0, ln)], acc.at[pl.ds(sl + off, ln)])


def _dump_acc_slice(acc, pa, sl, out_ref, c):
    for off, ln in PIECES:
        pltpu.sync_copy(acc.at[pl.ds(sl + off, ln)], pa.at[pl.ds(0, ln)])
        pltpu.sync_copy(pa.at[pl.ds(0, ln)],
                        out_ref.at[pl.ds(c * NPAD + sl + off, ln)])


def _layer1_body(sub_ref, feat_ref, norm_ref, out_ref,
                 table_sp, acc, pa, pb, src_b, dst_b, val_b,
                 sem_i0, sem_i1, sem_g, sem_s):
    c = lax.axis_index("c")
    s = lax.axis_index("s")
    w = c * 16 + s
    sl = s * SLICE
    for off, ln in PIECES:
        pltpu.sync_copy(feat_ref.at[pl.ds(sl + off, ln)], pa.at[pl.ds(0, ln)])
        pltpu.sync_copy(norm_ref.at[pl.ds(sl + off, ln)], pb.at[pl.ds(0, ln)])

        def mbody(j, _):
            ix = pl.ds(j * 16, 16)
            pa[ix] = pa[ix] * pb[ix]
            return 0

        lax.fori_loop(0, ln // 16, mbody, 0)
        pltpu.sync_copy(pa.at[pl.ds(0, ln)], table_sp.at[pl.ds(sl + off, ln)])
    _zero_acc_slice(acc, pa, sl)
    plsc.subcore_barrier()

    _edge_pass(sub_ref, table_sp, acc, src_b, dst_b, val_b,
               sem_i0, sem_i1, sem_g, sem_s, w)

    plsc.subcore_barrier()
    _dump_acc_slice(acc, pa, sl, out_ref, c)


def _layer2_body(sub_ref, p_ref, norm_ref, w_ref, out_ref,
                 table_sp, acc, pa, pb, pc, wb, src_b, dst_b, val_b,
                 sem_i0, sem_i1, sem_g, sem_s):
    c = lax.axis_index("c")
    s = lax.axis_index("s")
    w = c * 16 + s
    sl = s * SLICE
    pltpu.sync_copy(w_ref, wb)
    wv = [wb[pl.ds(k * 16, 16)] for k in range(9)]
    for off, ln in PIECES:
        pltpu.sync_copy(p_ref.at[pl.ds(sl + off, ln)], pa.at[pl.ds(0, ln)])
        pltpu.sync_copy(p_ref.at[pl.ds(NPAD + sl + off, ln)],
                        pb.at[pl.ds(0, ln)])
        pltpu.sync_copy(norm_ref.at[pl.ds(sl + off, ln)], pc.at[pl.ds(0, ln)])

        def sbody(j, _):
            ix = pl.ds(j * 16, 16)
            nv = pc[ix]
            t = (pa[ix] + pb[ix]) * nv
            sv = jnp.zeros((16,), jnp.float32)
            for k in range(3):
                sv = sv + jnp.maximum(t * wv[k] + wv[3 + k], 0.0) * wv[6 + k]
            pa[ix] = sv * nv
            return 0

        lax.fori_loop(0, ln // 16, sbody, 0)
        pltpu.sync_copy(pa.at[pl.ds(0, ln)], table_sp.at[pl.ds(sl + off, ln)])
    _zero_acc_slice(acc, pa, sl)
    plsc.subcore_barrier()

    _edge_pass(sub_ref, table_sp, acc, src_b, dst_b, val_b,
               sem_i0, sem_i1, sem_g, sem_s, w)

    plsc.subcore_barrier()
    _dump_acc_slice(acc, pa, sl, out_ref, c)


def _fin_body(p_ref, n_ref, b_ref, o_ref):
    o_ref[...] = jnp.maximum((p_ref[0] + p_ref[1]) * n_ref[...] + b_ref[...], 0.0)


_finalize = pl.pallas_call(
    _fin_body,
    out_shape=jax.ShapeDtypeStruct((NPAD // 128, 128), jnp.float32),
)


@functools.lru_cache(maxsize=None)
def _build_layers():
    common = [
        pltpu.MemorySpace.VMEM_SHARED((NPAD,), jnp.float32),
        pltpu.MemorySpace.VMEM_SHARED((NPAD,), jnp.float32),
        pltpu.MemorySpace.VMEM((PBUF,), jnp.float32),
        pltpu.MemorySpace.VMEM((PBUF,), jnp.float32),
    ]
    bufs = [
        pltpu.MemorySpace.VMEM((2, CH), jnp.int32),
        pltpu.MemorySpace.VMEM((2, CH), jnp.int32),
        pltpu.MemorySpace.VMEM((2, CH), jnp.float32),
        pltpu.SemaphoreType.DMA,
        pltpu.SemaphoreType.DMA,
        pltpu.SemaphoreType.DMA,
        pltpu.SemaphoreType.DMA,
    ]
    cp = pltpu.CompilerParams(use_tc_tiling_on_sc=False)
    l1 = pl.kernel(
        _layer1_body,
        compiler_params=cp,
        out_type=jax.ShapeDtypeStruct((2 * NPAD,), jnp.float32),
        mesh=_mesh(),
        scratch_types=common + bufs,
    )
    l2 = pl.kernel(
        _layer2_body,
        compiler_params=cp,
        out_type=jax.ShapeDtypeStruct((2 * NPAD,), jnp.float32),
        mesh=_mesh(),
        scratch_types=common + [
            pltpu.MemorySpace.VMEM((PBUF,), jnp.float32),
            pltpu.MemorySpace.VMEM((144,), jnp.float32),
        ] + bufs,
    )
    return l1, l2


def kernel(subgraph, feat, norm, W1, b1, W2, b2):
    _l1, _l2 = _build_layers()
    featp = jnp.pad(feat.reshape(N), (0, NPAD - N))
    normp = jnp.pad(norm.reshape(N), (0, NPAD - N))
    p1 = _l1(subgraph, featp, normp)
    wtab = jnp.concatenate([
        jnp.broadcast_to(W1.reshape(3, 1), (3, 16)).reshape(-1),
        jnp.broadcast_to(b1.reshape(3, 1), (3, 16)).reshape(-1),
        jnp.broadcast_to(W2.reshape(3, 1), (3, 16)).reshape(-1),
    ])
    p2 = _l2(subgraph, p1, normp, wtab)
    b2t = jnp.broadcast_to(b2.reshape(1, 1), (1, 1))
    out = _finalize(p2.reshape(2, NPAD // 128, 128),
                    normp.reshape(NPAD // 128, 128), b2t)
    return out.reshape(NPAD)[:N].reshape(N, 1)


# speedup vs baseline: 1.0472x; 1.0282x over previous
import functools

import jax
import jax.numpy as jnp
from jax import lax
from jax.experimental import pallas as pl
from jax.experimental.pallas import tpu as pltpu
from jax.experimental.pallas import tpu_sc as plsc

N = 100000
E = 6400000
NPAD = 100096
SLICE = NPAD // 16
PIECES = ((0, 1568), (1568, 1568), (3136, 1568), (4704, 1552))
PBUF = 1568
G = 120
NCHUNK = 13
CH = G * 128


@functools.lru_cache(maxsize=None)
def _mesh():
    return plsc.VectorSubcoreMesh(core_axis_name="c", subcore_axis_name="s",
                                  num_cores=2, num_subcores=16)


def _stage_copies(sub_ref, src_b, dst_b, sem, w, i, p):
    gbase = (w * 195 + jnp.minimum(w, 10)) * 8
    e0 = (gbase + i * G) * 128
    return (pltpu.make_async_copy(sub_ref.at[0, pl.ds(e0, CH)],
                                  src_b.at[p], sem),
            pltpu.make_async_copy(sub_ref.at[1, pl.ds(e0, CH)],
                                  dst_b.at[p], sem))


def _edge_pass(sub_ref, table, acc, src_b, dst_b, val_b,
               sem_i0, sem_i1, sem_g, sem_s, w):
    gbase = (w * 195 + jnp.minimum(w, 10)) * 8

    def idx_copies(i, p):
        return _stage_copies(sub_ref, src_b, dst_b,
                             sem_i0 if p == 0 else sem_i1, w, i, p)

    def gather_copy(p):
        return pltpu.make_async_copy(table.at[src_b.at[p]], val_b.at[p],
                                     sem_g)

    def scatter_copy(p):
        return pltpu.make_async_copy(val_b.at[p], acc.at[dst_b.at[p]],
                                     sem_s)

    for i in range(NCHUNK):
        p = i % 2
        for cpy in idx_copies(i, p):
            cpy.wait()
        gather_copy(p).start()
        if i >= 1:
            scatter_copy(1 - p).wait()
        if i + 1 < NCHUNK:
            for cpy in idx_copies(i + 1, 1 - p):
                cpy.start()
        gather_copy(p).wait()
        pltpu.async_copy(val_b.at[p], acc.at[dst_b.at[p]], sem_s, add=True)
    scatter_copy((NCHUNK - 1) % 2).wait()

    @pl.when(w < 10)
    def _tail():
        e0 = (gbase + NCHUNK * G) * 128
        pltpu.sync_copy(sub_ref.at[0, pl.ds(e0, 1024)],
                        src_b.at[0, pl.ds(0, 1024)])
        pltpu.sync_copy(sub_ref.at[1, pl.ds(e0, 1024)],
                        dst_b.at[0, pl.ds(0, 1024)])
        pltpu.async_copy(table.at[src_b.at[0, pl.ds(0, 1024)]],
                         val_b.at[0, pl.ds(0, 1024)], sem_g).wait()
        pltpu.sync_copy(val_b.at[0, pl.ds(0, 1024)],
                        acc.at[dst_b.at[0, pl.ds(0, 1024)]], add=True)


def _zero_acc_slice(acc, pa, sl):
    def zbody(j, _):
        pa[pl.ds(j * 16, 16)] = jnp.zeros((16,), jnp.float32)
        return 0

    lax.fori_loop(0, PBUF // 16, zbody, 0)
    for off, ln in PIECES:
        pltpu.sync_copy(pa.at[pl.ds(0, ln)], acc.at[pl.ds(sl + off, ln)])


def _dump_acc_slice(acc, pa, sl, out_ref, c):
    for off, ln in PIECES:
        pltpu.sync_copy(acc.at[pl.ds(sl + off, ln)], pa.at[pl.ds(0, ln)])
        pltpu.sync_copy(pa.at[pl.ds(0, ln)],
                        out_ref.at[pl.ds(c * NPAD + sl + off, ln)])


def _layer1_body(sub_ref, feat_ref, norm_ref, out_ref,
                 table_sp, acc, pa, pb, src_b, dst_b, val_b,
                 sem_i0, sem_i1, sem_g, sem_s):
    c = lax.axis_index("c")
    s = lax.axis_index("s")
    w = c * 16 + s
    sl = s * SLICE
    for cpy in _stage_copies(sub_ref, src_b, dst_b, sem_i0, w, 0, 0):
        cpy.start()
    for off, ln in PIECES:
        cps = (pltpu.make_async_copy(feat_ref.at[pl.ds(sl + off, ln)],
                                     pa.at[pl.ds(0, ln)], sem_i1),
               pltpu.make_async_copy(norm_ref.at[pl.ds(sl + off, ln)],
                                     pb.at[pl.ds(0, ln)], sem_i1))
        for cpy in cps:
            cpy.start()
        for cpy in cps:
            cpy.wait()

        def mbody(j, _):
            ix = pl.ds(j * 16, 16)
            pa[ix] = pa[ix] * pb[ix]
            return 0

        lax.fori_loop(0, ln // 16, mbody, 0)
        pltpu.sync_copy(pa.at[pl.ds(0, ln)], table_sp.at[pl.ds(sl + off, ln)])
    _zero_acc_slice(acc, pa, sl)
    plsc.subcore_barrier()

    _edge_pass(sub_ref, table_sp, acc, src_b, dst_b, val_b,
               sem_i0, sem_i1, sem_g, sem_s, w)

    plsc.subcore_barrier()
    _dump_acc_slice(acc, pa, sl, out_ref, c)


def _layer2_body(sub_ref, p_ref, norm_ref, w_ref, out_ref,
                 table_sp, acc, pa, pb, pc, wb, src_b, dst_b, val_b,
                 sem_i0, sem_i1, sem_g, sem_s):
    c = lax.axis_index("c")
    s = lax.axis_index("s")
    w = c * 16 + s
    sl = s * SLICE
    for cpy in _stage_copies(sub_ref, src_b, dst_b, sem_i0, w, 0, 0):
        cpy.start()
    pltpu.sync_copy(w_ref, wb)
    wv = [wb[pl.ds(k * 16, 16)] for k in range(9)]
    for off, ln in PIECES:
        cps = (pltpu.make_async_copy(p_ref.at[pl.ds(sl + off, ln)],
                                     pa.at[pl.ds(0, ln)], sem_i1),
               pltpu.make_async_copy(p_ref.at[pl.ds(NPAD + sl + off, ln)],
                                     pb.at[pl.ds(0, ln)], sem_i1),
               pltpu.make_async_copy(norm_ref.at[pl.ds(sl + off, ln)],
                                     pc.at[pl.ds(0, ln)], sem_i1))
        for cpy in cps:
            cpy.start()
        for cpy in cps:
            cpy.wait()

        def sbody(j, _):
            ix = pl.ds(j * 16, 16)
            nv = pc[ix]
            t = (pa[ix] + pb[ix]) * nv
            sv = jnp.zeros((16,), jnp.float32)
            for k in range(3):
                sv = sv + jnp.maximum(t * wv[k] + wv[3 + k], 0.0) * wv[6 + k]
            pa[ix] = sv * nv
            return 0

        lax.fori_loop(0, ln // 16, sbody, 0)
        pltpu.sync_copy(pa.at[pl.ds(0, ln)], table_sp.at[pl.ds(sl + off, ln)])
    _zero_acc_slice(acc, pa, sl)
    plsc.subcore_barrier()

    _edge_pass(sub_ref, table_sp, acc, src_b, dst_b, val_b,
               sem_i0, sem_i1, sem_g, sem_s, w)

    plsc.subcore_barrier()
    _dump_acc_slice(acc, pa, sl, out_ref, c)


def _fin_body(p_ref, n_ref, b_ref, o_ref):
    o_ref[...] = jnp.maximum((p_ref[0] + p_ref[1]) * n_ref[...] + b_ref[...], 0.0)


_finalize = pl.pallas_call(
    _fin_body,
    out_shape=jax.ShapeDtypeStruct((NPAD // 128, 128), jnp.float32),
)


@functools.lru_cache(maxsize=None)
def _build_layers():
    common = [
        pltpu.MemorySpace.VMEM_SHARED((NPAD,), jnp.float32),
        pltpu.MemorySpace.VMEM_SHARED((NPAD,), jnp.float32),
        pltpu.MemorySpace.VMEM((PBUF,), jnp.float32),
        pltpu.MemorySpace.VMEM((PBUF,), jnp.float32),
    ]
    bufs = [
        pltpu.MemorySpace.VMEM((2, CH), jnp.int32),
        pltpu.MemorySpace.VMEM((2, CH), jnp.int32),
        pltpu.MemorySpace.VMEM((2, CH), jnp.float32),
        pltpu.SemaphoreType.DMA,
        pltpu.SemaphoreType.DMA,
        pltpu.SemaphoreType.DMA,
        pltpu.SemaphoreType.DMA,
    ]
    cp = pltpu.CompilerParams(use_tc_tiling_on_sc=False)
    l1 = pl.kernel(
        _layer1_body,
        compiler_params=cp,
        out_type=jax.ShapeDtypeStruct((2 * NPAD,), jnp.float32),
        mesh=_mesh(),
        scratch_types=common + bufs,
    )
    l2 = pl.kernel(
        _layer2_body,
        compiler_params=cp,
        out_type=jax.ShapeDtypeStruct((2 * NPAD,), jnp.float32),
        mesh=_mesh(),
        scratch_types=common + [
            pltpu.MemorySpace.VMEM((PBUF,), jnp.float32),
            pltpu.MemorySpace.VMEM((144,), jnp.float32),
        ] + bufs,
    )
    return l1, l2


def kernel(subgraph, feat, norm, W1, b1, W2, b2):
    _l1, _l2 = _build_layers()
    featp = jnp.pad(feat.reshape(N), (0, NPAD - N))
    normp = jnp.pad(norm.reshape(N), (0, NPAD - N))
    p1 = _l1(subgraph, featp, normp)
    wtab = jnp.concatenate([
        jnp.broadcast_to(W1.reshape(3, 1), (3, 16)).reshape(-1),
        jnp.broadcast_to(b1.reshape(3, 1), (3, 16)).reshape(-1),
        jnp.broadcast_to(W2.reshape(3, 1), (3, 16)).reshape(-1),
    ])
    p2 = _l2(subgraph, p1, normp, wtab)
    b2t = jnp.broadcast_to(b2.reshape(1, 1), (1, 1))
    out = _finalize(p2.reshape(2, NPAD // 128, 128),
                    normp.reshape(NPAD // 128, 128), b2t)
    return out.reshape(NPAD)[:N].reshape(N, 1)
